# baseline (speedup 1.0000x reference)
import dataclasses
import functools

import jax
import jax.numpy as jnp
from jax import lax
from jax.experimental import pallas as pl
from jax.experimental.pallas import tpu as pltpu
from jax.experimental.pallas import tpu_sc as plsc

N_NODE = 10000
N_PAD = 10240
EMB = 128
LAYERS = 2
NNZ = 320000
NC = 2
NS = 16
ROWS_PER_TILE = N_PAD // NS
NNZ_PER_TILE = NNZ // NS
CH = 128
N_FULL = NNZ_PER_TILE // CH
TAIL = NNZ_PER_TILE - N_FULL * CH

def _vector_mesh():
    return plsc.VectorSubcoreMesh(
        core_axis_name="c", subcore_axis_name="s", num_cores=NC, num_subcores=NS)


_sc_params = pltpu.CompilerParams()
if "needs_layout_passes" in pltpu.CompilerParams.__dataclass_fields__:
    _sc_params = dataclasses.replace(_sc_params, needs_layout_passes=False)


def _scale_rows(rows_ref, val_ref, base, n_rows):
    @plsc.parallel_loop(0, n_rows, unroll=8)
    def _(r):
        vsp = plsc.load_gather(val_ref, [jnp.full((16,), base + r, jnp.int32)])
        for q in range(EMB // 16):
            sl = pl.ds(q * 16, 16)
            rows_ref[r, sl] = rows_ref[r, sl] * vsp


def _spmm_body(table_hbm, idx_hbm, val_hbm, out_hbm,
               d0, d1, d2, s0, s1, s2, v0, v1, v2, b0, b1,
               dst_t, src_t, val_t, rows_t,
               gs0, gs1, is0, is1, is2, ss0, ss1, acc_sh):
    c = lax.axis_index("c")
    s = lax.axis_index("s")

    dbufs, sbufs, vbufs = (d0, d1, d2), (s0, s1, s2), (v0, v1, v2)
    bufs = (b0, b1)
    gsems = (gs0, gs1)
    isems = (is0, is1, is2)
    ssems = (ss0, ss1)

    @pl.loop(0, CH)
    def _(r):
        for q in range(EMB // 16):
            b0[r, pl.ds(q * 16, 16)] = jnp.zeros((16,), jnp.float32)

    row0 = s * ROWS_PER_TILE
    for k in range(ROWS_PER_TILE // CH):
        pltpu.sync_copy(b0, acc_sh.at[pl.ds(row0 + k * CH, CH)])
    plsc.subcore_barrier()

    dst_off = (2 * c) * NNZ + s * NNZ_PER_TILE
    src_off = (2 * c + 1) * NNZ + s * NNZ_PER_TILE
    val_off = c * NNZ + s * NNZ_PER_TILE

    def start_idx_fetch(g, k):
        pltpu.async_copy(idx_hbm.at[pl.ds(dst_off + g * CH, CH)], dbufs[k],
                         isems[k])
        pltpu.async_copy(idx_hbm.at[pl.ds(src_off + g * CH, CH)], sbufs[k],
                         isems[k])
        pltpu.async_copy(val_hbm.at[pl.ds(val_off + g * CH, CH)], vbufs[k],
                         isems[k])

    def wait_idx_fetch(g, k):
        pltpu.make_async_copy(idx_hbm.at[pl.ds(dst_off + g * CH, CH)],
                              dbufs[k], isems[k]).wait()
        pltpu.make_async_copy(idx_hbm.at[pl.ds(src_off + g * CH, CH)],
                              sbufs[k], isems[k]).wait()
        pltpu.make_async_copy(val_hbm.at[pl.ds(val_off + g * CH, CH)],
                              vbufs[k], isems[k]).wait()

    def start_gather(j, k):
        pltpu.async_copy(table_hbm.at[sbufs[k]], bufs[j], gsems[j])

    def wait_gather(j, k):
        pltpu.make_async_copy(table_hbm.at[sbufs[k]], bufs[j], gsems[j]).wait()

    def start_scatter(j, k):
        pltpu.async_copy(bufs[j], acc_sh.at[dbufs[k]], ssems[j], add=True)

    def wait_scatter(j, k):
        pltpu.make_async_copy(bufs[j], acc_sh.at[dbufs[k]], ssems[j]).wait()

    start_idx_fetch(0, 0)
    wait_idx_fetch(0, 0)
    start_gather(0, 0)
    start_idx_fetch(1, 1)

    @pl.loop(0, N_FULL // 6)
    def _(h):
        for u in range(6):
            g = 6 * h + u
            j, jn = u % 2, 1 - (u % 2)
            k, k1, k2 = u % 3, (u + 1) % 3, (u + 2) % 3
            wait_gather(j, k)

            @pl.when(g + 1 < N_FULL)
            def _():
                wait_idx_fetch(g + 1, k1)

                @pl.when(g >= 1)
                def _():
                    wait_scatter(jn, k2)
                start_gather(jn, k1)

            _scale_rows(bufs[j], vbufs[k], 0, CH)
            start_scatter(j, k)

            @pl.when(g + 2 < N_FULL)
            def _():
                start_idx_fetch(g + 2, k2)

    wait_scatter(0, 1)
    wait_scatter(1, 2)

    pltpu.sync_copy(idx_hbm.at[pl.ds(dst_off + N_FULL * CH, TAIL)], dst_t)
    pltpu.sync_copy(idx_hbm.at[pl.ds(src_off + N_FULL * CH, TAIL)], src_t)
    pltpu.sync_copy(val_hbm.at[pl.ds(val_off + N_FULL * CH, TAIL)], val_t)
    pltpu.async_copy(table_hbm.at[src_t], rows_t, gs0).wait()
    _scale_rows(rows_t, val_t, 0, TAIL)
    pltpu.sync_copy(rows_t, acc_sh.at[dst_t], add=True)

    plsc.subcore_barrier()
    for k in range(ROWS_PER_TILE // CH):
        pltpu.sync_copy(acc_sh.at[pl.ds(row0 + k * CH, CH)], b0)
        pltpu.sync_copy(b0, out_hbm.at[c, pl.ds(row0 + k * CH, CH)])


@functools.cache
def _make_spmm_sc():
  return pl.kernel(
    _spmm_body,
    out_type=jax.ShapeDtypeStruct((NC, N_PAD, EMB), jnp.float32),
    mesh=_vector_mesh(),
    scratch_types=(
        [pltpu.VMEM((CH,), jnp.int32)] * 6
        + [pltpu.VMEM((CH,), jnp.float32)] * 3
        + [pltpu.VMEM((CH, EMB), jnp.float32)] * 2
        + [pltpu.VMEM((TAIL,), jnp.int32)] * 2
        + [pltpu.VMEM((TAIL,), jnp.float32),
           pltpu.VMEM((TAIL, EMB), jnp.float32)]
        + [pltpu.SemaphoreType.DMA] * 7
        + [pltpu.VMEM_SHARED((N_PAD, EMB), jnp.float32)]
    ),
    compiler_params=_sc_params,
  )


def _gather_body(table_hbm, ids_hbm, out_hbm, ids_a, b0, b1,
                 gs0, gs1, ws0, ws1, *, n_ids):
    c = lax.axis_index("c")
    s = lax.axis_index("s")
    wid = s * NC + c
    per_tile = n_ids // (NC * NS)
    n_chunks = per_tile // CH
    base0 = wid * per_tile

    bufs = (b0, b1)
    gsems = (gs0, gs1)
    wsems = (ws0, ws1)

    pltpu.sync_copy(ids_hbm.at[pl.ds(base0, per_tile)], ids_a)

    def start_gather(g, j):
        pltpu.async_copy(table_hbm.at[ids_a.at[pl.ds(g * CH, CH)]],
                         bufs[j], gsems[j])

    def wait_gather(g, j):
        pltpu.make_async_copy(table_hbm.at[ids_a.at[pl.ds(g * CH, CH)]],
                              bufs[j], gsems[j]).wait()

    def start_wb(g, j):
        pltpu.async_copy(bufs[j], out_hbm.at[pl.ds(base0 + g * CH, CH)],
                         wsems[j])

    def wait_wb(g, j):
        pltpu.make_async_copy(bufs[j], out_hbm.at[pl.ds(base0 + g * CH, CH)],
                              wsems[j]).wait()

    start_gather(0, 0)

    @pl.loop(0, n_chunks // 2)
    def _(h):
        for j in range(2):
            g = 2 * h + j
            jn = 1 - j
            wait_gather(g, j)

            @pl.when(g + 1 < n_chunks)
            def _():
                @pl.when(g >= 1)
                def _():
                    wait_wb(g - 1, jn)
                start_gather(g + 1, jn)

            start_wb(g, j)

    wait_wb(n_chunks - 2, 0)
    wait_wb(n_chunks - 1, 1)


@functools.cache
def _make_gather_sc(n_ids):
    per_tile = n_ids // (NC * NS)
    return pl.kernel(
        functools.partial(_gather_body, n_ids=n_ids),
        out_type=jax.ShapeDtypeStruct((n_ids, EMB), jnp.float32),
        mesh=_vector_mesh(),
        scratch_types=[
            pltpu.VMEM((per_tile,), jnp.int32),
            pltpu.VMEM((CH, EMB), jnp.float32),
            pltpu.VMEM((CH, EMB), jnp.float32),
            pltpu.SemaphoreType.DMA,
            pltpu.SemaphoreType.DMA,
            pltpu.SemaphoreType.DMA,
            pltpu.SemaphoreType.DMA,
        ],
        compiler_params=_sc_params,
    )


_RB = 2048


def _gated_body(x_ref, w0_ref, b0_ref, w1_ref, b1_ref, o_ref):
    x = x_ref[...]
    g0 = jax.nn.sigmoid(jnp.dot(x, w0_ref[...], preferred_element_type=jnp.float32)
                        + b0_ref[...])
    g1 = jax.nn.sigmoid(jnp.dot(x, w1_ref[...], preferred_element_type=jnp.float32)
                        + b1_ref[...])
    o_ref[0] = x * g0
    o_ref[1] = x * g1


def _gated_tc(user_emb, W0, b0, W1, b1):
    return pl.pallas_call(
        _gated_body,
        grid=(N_PAD // _RB,),
        in_specs=[
            pl.BlockSpec((_RB, EMB), lambda i: (i, 0)),
            pl.BlockSpec((EMB, EMB), lambda i: (0, 0)),
            pl.BlockSpec((1, EMB), lambda i: (0, 0)),
            pl.BlockSpec((EMB, EMB), lambda i: (0, 0)),
            pl.BlockSpec((1, EMB), lambda i: (0, 0)),
        ],
        out_specs=pl.BlockSpec((2, _RB, EMB), lambda i: (0, i, 0)),
        out_shape=jax.ShapeDtypeStruct((2, N_PAD, EMB), jnp.float32),
    )(user_emb, W0, b0, W1, b1)


def _norm_acc_body(acc_ref, u_ref, o_ref):
    u = u_ref[...]
    n = jnp.sqrt(jnp.sum(u * u, axis=-1, keepdims=True))
    o_ref[...] = acc_ref[...] + u / jnp.maximum(n, 1e-12)


def _norm_acc_tc(acc, u_new):
    spec = pl.BlockSpec((2, _RB, EMB), lambda i: (0, i, 0))
    return pl.pallas_call(
        _norm_acc_body,
        grid=(N_PAD // _RB,),
        in_specs=[spec, spec],
        out_specs=spec,
        out_shape=jax.ShapeDtypeStruct((2, N_PAD, EMB), jnp.float32),
    )(acc, u_new)


def _combine_body(acc_ref, att_ref, attm_ref, o_ref):
    a2 = acc_ref[0]
    a3 = acc_ref[1]
    att = att_ref[...]
    attm = attm_ref[...]
    w2 = jnp.sum(att * jnp.dot(a2, attm, preferred_element_type=jnp.float32),
                 axis=1, keepdims=True)
    w3 = jnp.sum(att * jnp.dot(a3, attm, preferred_element_type=jnp.float32),
                 axis=1, keepdims=True)
    m = jnp.maximum(w2, w3)
    e2 = jnp.exp(w2 - m)
    e3 = jnp.exp(w3 - m)
    inv = 1.0 / (e2 + e3)
    o_ref[...] = (e2 * inv) * a2 + (e3 * inv) * a3


def _combine_tc(acc, att, att_m):
    return pl.pallas_call(
        _combine_body,
        grid=(N_PAD // _RB,),
        in_specs=[
            pl.BlockSpec((2, _RB, EMB), lambda i: (0, i, 0)),
            pl.BlockSpec((1, EMB), lambda i: (0, 0)),
            pl.BlockSpec((EMB, EMB), lambda i: (0, 0)),
        ],
        out_specs=pl.BlockSpec((_RB, EMB), lambda i: (i, 0)),
        out_shape=jax.ShapeDtypeStruct((N_PAD, EMB), jnp.float32),
    )(acc, att, att_m)


_BB = 8


def _cascade_att_body(ids_ref, cas_ref, o_ref):
    cas = cas_ref[...]
    ids = ids_ref[...]
    src = cas[:, 0:1, :]
    sc = jnp.sum(cas * src, axis=2) * (1.0 / jnp.sqrt(jnp.float32(EMB)))
    sc = jnp.where(ids == 0, jnp.float32(-1e9), sc)
    m = jnp.max(sc, axis=1, keepdims=True)
    e = jnp.exp(sc - m)
    attn = e / jnp.sum(e, axis=1, keepdims=True)
    ctx = jnp.sum(attn[:, :, None] * cas, axis=1, keepdims=True)
    o_ref[...] = cas + ctx


def _cascade_att_tc(ids, cas):
    b, l, d = cas.shape
    return pl.pallas_call(
        _cascade_att_body,
        grid=(b // _BB,),
        in_specs=[
            pl.BlockSpec((_BB, l), lambda i: (i, 0)),
            pl.BlockSpec((_BB, l, d), lambda i: (i, 0, 0)),
        ],
        out_specs=pl.BlockSpec((_BB, l, d), lambda i: (i, 0, 0)),
        out_shape=jax.ShapeDtypeStruct((b, l, d), jnp.float32),
    )(ids, cas)


def kernel(input, H_item_idx, H_item_val, H_user_idx, H_user_val,
           user_emb, W0, b0, W1, b1, att, att_m):
    ids = input.astype(jnp.int32)
    b, l = ids.shape

    hi = H_item_idx.astype(jnp.int32)
    hu = H_user_idx.astype(jnp.int32)
    idx_all = jnp.concatenate(
        [hi, hu + jnp.array([[0], [N_PAD]], jnp.int32)],
        axis=0).reshape(-1)
    val_all = jnp.concatenate([H_item_val, H_user_val])

    emb_pad = jnp.pad(user_emb, ((0, N_PAD - N_NODE), (0, 0)))
    u_cur = _gated_tc(emb_pad, W0, b0, W1, b1)

    def layer(carry, _):
        u, a = carry
        u = _make_spmm_sc()(u.reshape(2 * N_PAD, EMB), idx_all, val_all)
        return (u, _norm_acc_tc(a, u)), None

    (_, acc), _ = lax.scan(layer, (u_cur, u_cur), None, length=LAYERS)

    hg = _combine_tc(acc, att, att_m)

    cas_flat = _make_gather_sc(b * l)(hg, ids.reshape(-1))
    cas = cas_flat.reshape(b, l, EMB)
    l_cas = _cascade_att_tc(ids, cas)
    return (l_cas, cas)


# speedup vs baseline: 1.0615x; 1.0615x over previous
import dataclasses
import functools

import jax
import jax.numpy as jnp
from jax import lax
from jax.experimental import pallas as pl
from jax.experimental.pallas import tpu as pltpu
from jax.experimental.pallas import tpu_sc as plsc

N_NODE = 10000
N_PAD = 10240
EMB = 128
LAYERS = 2
NNZ = 320000
NC = 2
NS = 16
ROWS_PER_TILE = N_PAD // NS
NNZ_PER_TILE = NNZ // NS
CH = 128
N_FULL = NNZ_PER_TILE // CH
TAIL = NNZ_PER_TILE - N_FULL * CH

def _vector_mesh():
    return plsc.VectorSubcoreMesh(
        core_axis_name="c", subcore_axis_name="s", num_cores=NC, num_subcores=NS)


_sc_params = pltpu.CompilerParams()
if "needs_layout_passes" in pltpu.CompilerParams.__dataclass_fields__:
    _sc_params = dataclasses.replace(_sc_params, needs_layout_passes=False)


def _scale_rows(rows_ref, val_ref, base, n_rows):
    @plsc.parallel_loop(0, n_rows, unroll=4)
    def _(r):
        vsp = plsc.load_gather(val_ref, [jnp.full((16,), base + r, jnp.int32)])
        for q in range(EMB // 16):
            sl = pl.ds(q * 16, 16)
            rows_ref[r, sl] = rows_ref[r, sl] * vsp


def _spmm_body(table_hbm, idx_hbm, val_hbm, out_hbm,
               d0, d1, d2, s0, s1, s2, v0, v1, v2, b0, b1,
               dst_t, src_t, val_t, rows_t,
               gs0, gs1, is0, is1, is2, ss0, ss1, acc_sh):
    c = lax.axis_index("c")
    s = lax.axis_index("s")

    dbufs, sbufs, vbufs = (d0, d1, d2), (s0, s1, s2), (v0, v1, v2)
    bufs = (b0, b1)
    gsems = (gs0, gs1)
    isems = (is0, is1, is2)
    ssems = (ss0, ss1)

    @pl.loop(0, CH)
    def _(r):
        for q in range(EMB // 16):
            b0[r, pl.ds(q * 16, 16)] = jnp.zeros((16,), jnp.float32)

    row0 = s * ROWS_PER_TILE
    for k in range(ROWS_PER_TILE // CH):
        pltpu.sync_copy(b0, acc_sh.at[pl.ds(row0 + k * CH, CH)])
    plsc.subcore_barrier()

    dst_off = (2 * c) * NNZ + s * NNZ_PER_TILE
    src_off = (2 * c + 1) * NNZ + s * NNZ_PER_TILE
    val_off = c * NNZ + s * NNZ_PER_TILE

    def start_idx_fetch(g, k):
        pltpu.async_copy(idx_hbm.at[pl.ds(dst_off + g * CH, CH)], dbufs[k],
                         isems[k])
        pltpu.async_copy(idx_hbm.at[pl.ds(src_off + g * CH, CH)], sbufs[k],
                         isems[k])
        pltpu.async_copy(val_hbm.at[pl.ds(val_off + g * CH, CH)], vbufs[k],
                         isems[k])

    def wait_idx_fetch(g, k):
        pltpu.make_async_copy(idx_hbm.at[pl.ds(dst_off + g * CH, CH)],
                              dbufs[k], isems[k]).wait()
        pltpu.make_async_copy(idx_hbm.at[pl.ds(src_off + g * CH, CH)],
                              sbufs[k], isems[k]).wait()
        pltpu.make_async_copy(val_hbm.at[pl.ds(val_off + g * CH, CH)],
                              vbufs[k], isems[k]).wait()

    def start_gather(j, k):
        pltpu.async_copy(table_hbm.at[sbufs[k]], bufs[j], gsems[j])

    def wait_gather(j, k):
        pltpu.make_async_copy(table_hbm.at[sbufs[k]], bufs[j], gsems[j]).wait()

    def start_scatter(j, k):
        pltpu.async_copy(bufs[j], acc_sh.at[dbufs[k]], ssems[j], add=True)

    def wait_scatter(j, k):
        pltpu.make_async_copy(bufs[j], acc_sh.at[dbufs[k]], ssems[j]).wait()

    start_idx_fetch(0, 0)
    wait_idx_fetch(0, 0)
    start_gather(0, 0)
    start_idx_fetch(1, 1)

    @pl.loop(0, N_FULL // 6)
    def _(h):
        for u in range(6):
            g = 6 * h + u
            j, jn = u % 2, 1 - (u % 2)
            k, k1, k2 = u % 3, (u + 1) % 3, (u + 2) % 3
            wait_gather(j, k)

            @pl.when(g + 1 < N_FULL)
            def _():
                wait_idx_fetch(g + 1, k1)

                @pl.when(g >= 1)
                def _():
                    wait_scatter(jn, k2)
                start_gather(jn, k1)

            _scale_rows(bufs[j], vbufs[k], 0, CH)
            start_scatter(j, k)

            @pl.when(g + 2 < N_FULL)
            def _():
                start_idx_fetch(g + 2, k2)

    wait_scatter(0, 1)
    wait_scatter(1, 2)

    pltpu.sync_copy(idx_hbm.at[pl.ds(dst_off + N_FULL * CH, TAIL)], dst_t)
    pltpu.sync_copy(idx_hbm.at[pl.ds(src_off + N_FULL * CH, TAIL)], src_t)
    pltpu.sync_copy(val_hbm.at[pl.ds(val_off + N_FULL * CH, TAIL)], val_t)
    pltpu.async_copy(table_hbm.at[src_t], rows_t, gs0).wait()
    _scale_rows(rows_t, val_t, 0, TAIL)
    pltpu.sync_copy(rows_t, acc_sh.at[dst_t], add=True)

    plsc.subcore_barrier()
    for k in range(ROWS_PER_TILE // CH):
        pltpu.sync_copy(acc_sh.at[pl.ds(row0 + k * CH, CH)], b0)
        pltpu.sync_copy(b0, out_hbm.at[c, pl.ds(row0 + k * CH, CH)])


@functools.cache
def _make_spmm_sc():
  return pl.kernel(
    _spmm_body,
    out_type=jax.ShapeDtypeStruct((NC, N_PAD, EMB), jnp.float32),
    mesh=_vector_mesh(),
    scratch_types=(
        [pltpu.VMEM((CH,), jnp.int32)] * 6
        + [pltpu.VMEM((CH,), jnp.float32)] * 3
        + [pltpu.VMEM((CH, EMB), jnp.float32)] * 2
        + [pltpu.VMEM((TAIL,), jnp.int32)] * 2
        + [pltpu.VMEM((TAIL,), jnp.float32),
           pltpu.VMEM((TAIL, EMB), jnp.float32)]
        + [pltpu.SemaphoreType.DMA] * 7
        + [pltpu.VMEM_SHARED((N_PAD, EMB), jnp.float32)]
    ),
    compiler_params=_sc_params,
  )


def _gather_body(table_hbm, ids_hbm, out_hbm, ids_a, b0, b1,
                 gs0, gs1, ws0, ws1, *, n_ids):
    c = lax.axis_index("c")
    s = lax.axis_index("s")
    wid = s * NC + c
    per_tile = n_ids // (NC * NS)
    n_chunks = per_tile // CH
    base0 = wid * per_tile

    bufs = (b0, b1)
    gsems = (gs0, gs1)
    wsems = (ws0, ws1)

    pltpu.sync_copy(ids_hbm.at[pl.ds(base0, per_tile)], ids_a)

    def start_gather(g, j):
        pltpu.async_copy(table_hbm.at[ids_a.at[pl.ds(g * CH, CH)]],
                         bufs[j], gsems[j])

    def wait_gather(g, j):
        pltpu.make_async_copy(table_hbm.at[ids_a.at[pl.ds(g * CH, CH)]],
                              bufs[j], gsems[j]).wait()

    def start_wb(g, j):
        pltpu.async_copy(bufs[j], out_hbm.at[pl.ds(base0 + g * CH, CH)],
                         wsems[j])

    def wait_wb(g, j):
        pltpu.make_async_copy(bufs[j], out_hbm.at[pl.ds(base0 + g * CH, CH)],
                              wsems[j]).wait()

    start_gather(0, 0)

    @pl.loop(0, n_chunks // 2)
    def _(h):
        for j in range(2):
            g = 2 * h + j
            jn = 1 - j
            wait_gather(g, j)

            @pl.when(g + 1 < n_chunks)
            def _():
                @pl.when(g >= 1)
                def _():
                    wait_wb(g - 1, jn)
                start_gather(g + 1, jn)

            start_wb(g, j)

    wait_wb(n_chunks - 2, 0)
    wait_wb(n_chunks - 1, 1)


@functools.cache
def _make_gather_sc(n_ids):
    per_tile = n_ids // (NC * NS)
    return pl.kernel(
        functools.partial(_gather_body, n_ids=n_ids),
        out_type=jax.ShapeDtypeStruct((n_ids, EMB), jnp.float32),
        mesh=_vector_mesh(),
        scratch_types=[
            pltpu.VMEM((per_tile,), jnp.int32),
            pltpu.VMEM((CH, EMB), jnp.float32),
            pltpu.VMEM((CH, EMB), jnp.float32),
            pltpu.SemaphoreType.DMA,
            pltpu.SemaphoreType.DMA,
            pltpu.SemaphoreType.DMA,
            pltpu.SemaphoreType.DMA,
        ],
        compiler_params=_sc_params,
    )


_RB = 2048


def _gated_body(x_ref, w0_ref, b0_ref, w1_ref, b1_ref, o_ref):
    x = x_ref[...]
    g0 = jax.nn.sigmoid(jnp.dot(x, w0_ref[...], preferred_element_type=jnp.float32)
                        + b0_ref[...])
    g1 = jax.nn.sigmoid(jnp.dot(x, w1_ref[...], preferred_element_type=jnp.float32)
                        + b1_ref[...])
    o_ref[0] = x * g0
    o_ref[1] = x * g1


def _gated_tc(user_emb, W0, b0, W1, b1):
    return pl.pallas_call(
        _gated_body,
        grid=(N_PAD // _RB,),
        in_specs=[
            pl.BlockSpec((_RB, EMB), lambda i: (i, 0)),
            pl.BlockSpec((EMB, EMB), lambda i: (0, 0)),
            pl.BlockSpec((1, EMB), lambda i: (0, 0)),
            pl.BlockSpec((EMB, EMB), lambda i: (0, 0)),
            pl.BlockSpec((1, EMB), lambda i: (0, 0)),
        ],
        out_specs=pl.BlockSpec((2, _RB, EMB), lambda i: (0, i, 0)),
        out_shape=jax.ShapeDtypeStruct((2, N_PAD, EMB), jnp.float32),
    )(user_emb, W0, b0, W1, b1)


def _norm_acc_body(acc_ref, u_ref, o_ref):
    u = u_ref[...]
    n = jnp.sqrt(jnp.sum(u * u, axis=-1, keepdims=True))
    o_ref[...] = acc_ref[...] + u / jnp.maximum(n, 1e-12)


def _norm_acc_tc(acc, u_new):
    spec = pl.BlockSpec((2, _RB, EMB), lambda i: (0, i, 0))
    return pl.pallas_call(
        _norm_acc_body,
        grid=(N_PAD // _RB,),
        in_specs=[spec, spec],
        out_specs=spec,
        out_shape=jax.ShapeDtypeStruct((2, N_PAD, EMB), jnp.float32),
    )(acc, u_new)


def _combine_body(acc_ref, att_ref, attm_ref, o_ref):
    a2 = acc_ref[0]
    a3 = acc_ref[1]
    att = att_ref[...]
    attm = attm_ref[...]
    w2 = jnp.sum(att * jnp.dot(a2, attm, preferred_element_type=jnp.float32),
                 axis=1, keepdims=True)
    w3 = jnp.sum(att * jnp.dot(a3, attm, preferred_element_type=jnp.float32),
                 axis=1, keepdims=True)
    m = jnp.maximum(w2, w3)
    e2 = jnp.exp(w2 - m)
    e3 = jnp.exp(w3 - m)
    inv = 1.0 / (e2 + e3)
    o_ref[...] = (e2 * inv) * a2 + (e3 * inv) * a3


def _combine_tc(acc, att, att_m):
    return pl.pallas_call(
        _combine_body,
        grid=(N_PAD // _RB,),
        in_specs=[
            pl.BlockSpec((2, _RB, EMB), lambda i: (0, i, 0)),
            pl.BlockSpec((1, EMB), lambda i: (0, 0)),
            pl.BlockSpec((EMB, EMB), lambda i: (0, 0)),
        ],
        out_specs=pl.BlockSpec((_RB, EMB), lambda i: (i, 0)),
        out_shape=jax.ShapeDtypeStruct((N_PAD, EMB), jnp.float32),
    )(acc, att, att_m)


_BB = 16


def _cascade_att_body(ids_ref, cas_ref, o_ref):
    cas = cas_ref[...]
    ids = ids_ref[...]
    src = cas[:, 0:1, :]
    sc = jnp.sum(cas * src, axis=2) * (1.0 / jnp.sqrt(jnp.float32(EMB)))
    sc = jnp.where(ids == 0, jnp.float32(-1e9), sc)
    m = jnp.max(sc, axis=1, keepdims=True)
    e = jnp.exp(sc - m)
    attn = e / jnp.sum(e, axis=1, keepdims=True)
    ctx = jnp.sum(attn[:, :, None] * cas, axis=1, keepdims=True)
    o_ref[...] = cas + ctx


def _cascade_att_tc(ids, cas):
    b, l, d = cas.shape
    return pl.pallas_call(
        _cascade_att_body,
        grid=(b // _BB,),
        in_specs=[
            pl.BlockSpec((_BB, l), lambda i: (i, 0)),
            pl.BlockSpec((_BB, l, d), lambda i: (i, 0, 0)),
        ],
        out_specs=pl.BlockSpec((_BB, l, d), lambda i: (i, 0, 0)),
        out_shape=jax.ShapeDtypeStruct((b, l, d), jnp.float32),
    )(ids, cas)


def kernel(input, H_item_idx, H_item_val, H_user_idx, H_user_val,
           user_emb, W0, b0, W1, b1, att, att_m):
    ids = input.astype(jnp.int32)
    b, l = ids.shape

    hi = H_item_idx.astype(jnp.int32)
    hu = H_user_idx.astype(jnp.int32)
    idx_all = jnp.concatenate(
        [hi, hu + jnp.array([[0], [N_PAD]], jnp.int32)],
        axis=0).reshape(-1)
    val_all = jnp.concatenate([H_item_val, H_user_val])

    emb_pad = jnp.pad(user_emb, ((0, N_PAD - N_NODE), (0, 0)))
    u_cur = _gated_tc(emb_pad, W0, b0, W1, b1)

    def layer(carry, _):
        u, a = carry
        u = _make_spmm_sc()(u.reshape(2 * N_PAD, EMB), idx_all, val_all)
        return (u, _norm_acc_tc(a, u)), None

    (_, acc), _ = lax.scan(layer, (u_cur, u_cur), None, length=LAYERS)

    hg = _combine_tc(acc, att, att_m)

    cas_flat = _make_gather_sc(b * l)(hg, ids.reshape(-1))
    cas = cas_flat.reshape(b, l, EMB)
    l_cas = _cascade_att_tc(ids, cas)
    return (l_cas, cas)
